# baseline (speedup 1.0000x reference)
import functools

import jax
import jax.numpy as jnp
from jax import lax
from jax.experimental import pallas as pl
from jax.experimental.pallas import tpu as pltpu
from jax.experimental.pallas import tpu_sc as plsc

VOCAB = 1000000
NUM_PROMPT = 128
D = 32
B = 4096
H = 200

NC, NS, L = 2, 16, 16
NWB = 4
NWH = 8
BW = B // NWB
HW = H // NWH
GPER = 128
NG = BW // GPER


def _body(orig_hbm, new_t_hbm, idx_t_hbm, out_hbm,
          newt_v, idxf_v, safe_v, rows_v, trans_v, sem):
    wid = lax.axis_index("s") * NC + lax.axis_index("c")
    wh = wid // NWB
    wb = wid % NWB
    b0 = wb * BW
    h0 = wh * HW

    pltpu.sync_copy(new_t_hbm, newt_v)

    lanes = lax.iota(jnp.int32, L)

    def hrow(q, carry):
        h = h0 + q
        with jax.named_scope("ph_idx"):
            pltpu.sync_copy(idx_t_hbm.at[h, pl.ds(b0, BW)], idxf_v)

        with jax.named_scope("ph_clamp"):
            for c in range(BW // L):
                v = idxf_v[pl.ds(c * L, L)]
                safe_v[pl.ds(c * L, L)] = jnp.minimum(v, VOCAB - 1)

        with jax.named_scope("ph_gather"):
            descs = [
                pltpu.async_copy(
                    orig_hbm.at[safe_v.at[pl.ds(j * GPER, GPER)]],
                    rows_v.at[pl.ds(j * GPER, GPER)],
                    sem,
                )
                for j in range(NG)
            ]
            for dsc in descs:
                dsc.wait()

        def fix(t, c):
            v = idxf_v[pl.ds(t * L, L)]
            cnt = plsc.all_reduce_population_count(v >= VOCAB)

            @pl.when(cnt[0] > 0)
            def _():
                mask = v >= VOCAB
                pidx = jnp.maximum(v - VOCAB, 0)
                rowid = t * L + lanes
                for dd in range(D):
                    dvec = jnp.full((L,), dd, jnp.int32)
                    vals = plsc.load_gather(newt_v, [dvec, pidx])
                    plsc.store_scatter(rows_v, [rowid, dvec], vals, mask=mask)
            return c
        with jax.named_scope("ph_fix"):
            lax.fori_loop(0, BW // L, fix, 0)

        def tpose(g, c):
            rowid = g * L + lanes
            for dd in range(D):
                dvec = jnp.full((L,), dd, jnp.int32)
                vals = plsc.load_gather(rows_v, [rowid, dvec])
                trans_v[dd, pl.ds(g * L, L)] = vals
            return c
        with jax.named_scope("ph_tpose"):
            lax.fori_loop(0, BW // L, tpose, 0)

        with jax.named_scope("ph_wout"):
            pltpu.sync_copy(trans_v, out_hbm.at[h, :, pl.ds(b0, BW)])
        return carry

    lax.fori_loop(0, HW, hrow, 0)


@functools.partial(jax.jit, static_argnames=())
def _lookup(orig_weight, new_t, idx_t):
    mesh = plsc.VectorSubcoreMesh(core_axis_name="c", subcore_axis_name="s")
    f = pl.kernel(
        _body,
        out_type=jax.ShapeDtypeStruct((H, D, B), jnp.float32),
        mesh=mesh,
        scratch_types=[
            pltpu.VMEM((D, NUM_PROMPT), jnp.float32),
            pltpu.VMEM((BW,), jnp.int32),
            pltpu.VMEM((BW,), jnp.int32),
            pltpu.VMEM((BW, D), jnp.float32),
            pltpu.VMEM((D, BW), jnp.float32),
            pltpu.SemaphoreType.DMA,
        ],
        compiler_params=pltpu.CompilerParams(
            needs_layout_passes=False, use_tc_tiling_on_sc=False),
    )
    return f(orig_weight, new_t, idx_t)


def kernel(orig_weight, new_weight, input):
    idx_t = input.astype(jnp.int32).T
    new_t = new_weight.T
    out = _lookup(orig_weight, new_t, idx_t)
    return out.transpose(2, 0, 1)


# speedup vs baseline: 1.1799x; 1.1799x over previous
import functools

import jax
import jax.numpy as jnp
from jax import lax
from jax.experimental import pallas as pl
from jax.experimental.pallas import tpu as pltpu
from jax.experimental.pallas import tpu_sc as plsc

VOCAB = 1000000
NUM_PROMPT = 128
D = 32
B = 4096
H = 200

NC, NS, L = 2, 16, 16
NWB = 4
NWH = 8
BW = B // NWB
HW = H // NWH
GPER = 128
NG = BW // GPER


def _body(orig_hbm, new_t_hbm, idx_t_hbm, out_hbm,
          newt_v, idxf_v, safe_v, rows_v, trans_v, sem):
    wid = lax.axis_index("s") * NC + lax.axis_index("c")
    wh = wid // NWB
    wb = wid % NWB
    b0 = wb * BW
    h0 = wh * HW

    pltpu.sync_copy(new_t_hbm, newt_v)

    lanes = lax.iota(jnp.int32, L)

    def hrow(q, carry):
        h = h0 + q
        with jax.named_scope("ph_idx"):
            pltpu.sync_copy(idx_t_hbm.at[h, pl.ds(b0, BW)], idxf_v)

        with jax.named_scope("ph_clamp"):
            for c in range(BW // L):
                v = idxf_v[pl.ds(c * L, L)]
                safe_v[pl.ds(c * L, L)] = jnp.minimum(v, VOCAB - 1)

        with jax.named_scope("ph_gather"):
            descs = [
                pltpu.async_copy(
                    orig_hbm.at[safe_v.at[pl.ds(j * GPER, GPER)]],
                    rows_v.at[pl.ds(j * GPER, GPER)],
                    sem,
                )
                for j in range(NG)
            ]
            for dsc in descs:
                dsc.wait()

        def fix(t, c):
            v = idxf_v[pl.ds(t * L, L)]
            cnt = plsc.all_reduce_population_count(v >= VOCAB)

            @pl.when(cnt[0] > 0)
            def _():
                mask = v >= VOCAB
                pidx = jnp.maximum(v - VOCAB, 0)
                rowid = t * L + lanes
                for dd in range(D):
                    dvec = jnp.full((L,), dd, jnp.int32)
                    vals = plsc.load_gather(newt_v, [dvec, pidx])
                    plsc.store_scatter(rows_v, [rowid, dvec], vals, mask=mask)
            return c
        with jax.named_scope("ph_fix"):
            lax.fori_loop(0, BW // L, fix, 0)

        with jax.named_scope("ph_tpose"):
            @plsc.parallel_loop(0, BW // L, unroll=4)
            def tpose(g):
                rowid = g * L + lanes
                for dd in range(D):
                    dvec = jnp.full((L,), dd, jnp.int32)
                    vals = plsc.load_gather(rows_v, [rowid, dvec])
                    trans_v[dd, pl.ds(g * L, L)] = vals

        with jax.named_scope("ph_wout"):
            pltpu.sync_copy(trans_v, out_hbm.at[h, :, pl.ds(b0, BW)])
        return carry

    lax.fori_loop(0, HW, hrow, 0)


@functools.partial(jax.jit, static_argnames=())
def _lookup(orig_weight, new_t, idx_t):
    mesh = plsc.VectorSubcoreMesh(core_axis_name="c", subcore_axis_name="s")
    f = pl.kernel(
        _body,
        out_type=jax.ShapeDtypeStruct((H, D, B), jnp.float32),
        mesh=mesh,
        scratch_types=[
            pltpu.VMEM((D, NUM_PROMPT), jnp.float32),
            pltpu.VMEM((BW,), jnp.int32),
            pltpu.VMEM((BW,), jnp.int32),
            pltpu.VMEM((BW, D), jnp.float32),
            pltpu.VMEM((D, BW), jnp.float32),
            pltpu.SemaphoreType.DMA,
        ],
        compiler_params=pltpu.CompilerParams(
            needs_layout_passes=False, use_tc_tiling_on_sc=False),
    )
    return f(orig_weight, new_t, idx_t)


def kernel(orig_weight, new_weight, input):
    idx_t = input.astype(jnp.int32).T
    new_t = new_weight.T
    out = _lookup(orig_weight, new_t, idx_t)
    return out.transpose(2, 0, 1)


# speedup vs baseline: 1.3647x; 1.1566x over previous
import functools

import jax
import jax.numpy as jnp
from jax import lax
from jax.experimental import pallas as pl
from jax.experimental.pallas import tpu as pltpu
from jax.experimental.pallas import tpu_sc as plsc

VOCAB = 1000000
NUM_PROMPT = 128
D = 32
TOTAL = 4096 * 200

NC, NS, L = 2, 16, 16
NW = NC * NS
PER_W = TOTAL // NW
BLK = 1024
NBLK = PER_W // BLK
GPER = 128
NG = BLK // GPER


def _body(orig_hbm, new_hbm, idx_hbm, out_hbm,
          new_tab_v, idx0_v, idx1_v, safe0_v, safe1_v, rows0_v, rows1_v,
          sg0, sg1, so0, so1, si0, si1):
    wid = lax.axis_index("s") * NC + lax.axis_index("c")
    wbase = wid * PER_W

    idx_v = (idx0_v, idx1_v)
    safe_v = (safe0_v, safe1_v)
    rows_v = (rows0_v, rows1_v)
    sg = (sg0, sg1)
    so = (so0, so1)
    si = (si0, si1)

    pltpu.sync_copy(new_hbm, new_tab_v)

    def clamp_block(s):
        def clamp(t, m):
            v = idx_v[s][pl.ds(t * L, L)]
            safe_v[s][pl.ds(t * L, L)] = jnp.minimum(v, VOCAB - 1)
            return jnp.maximum(m, v)
        m = lax.fori_loop(0, BLK // L, clamp, jnp.zeros((L,), jnp.int32))
        return plsc.all_reduce_population_count(m >= VOCAB)[0]

    def fix_block(s, hits):
        @pl.when(hits > 0)
        def _():
            def fix(t, c):
                v = idx_v[s][pl.ds(t * L, L)]
                cnt = plsc.all_reduce_population_count(v >= VOCAB)

                @pl.when(cnt[0] > 0)
                def _():
                    mask = v >= VOCAB
                    pidx = jnp.maximum(v - VOCAB, 0)
                    rowid = t * L + lax.iota(jnp.int32, L)
                    for dd in range(D):
                        dvec = jnp.full((L,), dd, jnp.int32)
                        vals = plsc.load_gather(new_tab_v, [pidx, dvec])
                        plsc.store_scatter(rows_v[s], [rowid, dvec], vals,
                                           mask=mask)
                return c
            lax.fori_loop(0, BLK // L, fix, 0)

    def fire_gathers(s):
        return [
            pltpu.async_copy(
                orig_hbm.at[safe_v[s].at[pl.ds(j * GPER, GPER)]],
                rows_v[s].at[pl.ds(j * GPER, GPER)],
                sg[s],
            )
            for j in range(NG)
        ]

    out_desc = [None, None]
    gather_descs = None
    prev_hits = None

    idx_desc = [None, None]
    idx_desc[0] = pltpu.async_copy(
        idx_hbm.at[pl.ds(wbase, BLK)], idx_v[0], si[0])

    for b in range(NBLK):
        s = b % 2
        idx_desc[s].wait()
        hits = clamp_block(s)
        if out_desc[s] is not None:
            out_desc[s].wait()
        descs = fire_gathers(s)
        if b > 0:
            for dsc in gather_descs:
                dsc.wait()
            fix_block(1 - s, prev_hits)
            out_desc[1 - s] = pltpu.async_copy(
                rows_v[1 - s],
                out_hbm.at[pl.ds(wbase + (b - 1) * BLK, BLK)],
                so[1 - s],
            )
        if b + 1 < NBLK:
            idx_desc[1 - s] = pltpu.async_copy(
                idx_hbm.at[pl.ds(wbase + (b + 1) * BLK, BLK)],
                idx_v[1 - s], si[1 - s])
        gather_descs = descs
        prev_hits = hits

    s = (NBLK - 1) % 2
    for dsc in gather_descs:
        dsc.wait()
    fix_block(s, prev_hits)
    out_desc[s] = pltpu.async_copy(
        rows_v[s],
        out_hbm.at[pl.ds(wbase + (NBLK - 1) * BLK, BLK)],
        so[s],
    )
    out_desc[0].wait()
    out_desc[1].wait()


@functools.partial(jax.jit, static_argnames=())
def _lookup(orig_weight, new_weight, idx_flat):
    mesh = plsc.VectorSubcoreMesh(core_axis_name="c", subcore_axis_name="s")
    f = pl.kernel(
        _body,
        out_type=jax.ShapeDtypeStruct((TOTAL, D), jnp.float32),
        mesh=mesh,
        scratch_types=[
            pltpu.VMEM((NUM_PROMPT, D), jnp.float32),
            pltpu.VMEM((BLK,), jnp.int32),
            pltpu.VMEM((BLK,), jnp.int32),
            pltpu.VMEM((BLK,), jnp.int32),
            pltpu.VMEM((BLK,), jnp.int32),
            pltpu.VMEM((BLK, D), jnp.float32),
            pltpu.VMEM((BLK, D), jnp.float32),
            pltpu.SemaphoreType.DMA,
            pltpu.SemaphoreType.DMA,
            pltpu.SemaphoreType.DMA,
            pltpu.SemaphoreType.DMA,
            pltpu.SemaphoreType.DMA,
            pltpu.SemaphoreType.DMA,
        ],
        compiler_params=pltpu.CompilerParams(
            needs_layout_passes=False, use_tc_tiling_on_sc=False),
    )
    return f(orig_weight, new_weight, idx_flat)


def kernel(orig_weight, new_weight, input):
    idx_flat = input.reshape(-1).astype(jnp.int32)
    out = _lookup(orig_weight, new_weight, idx_flat)
    return out.reshape(input.shape + (D,))


# speedup vs baseline: 1.3686x; 1.0028x over previous
import functools

import jax
import jax.numpy as jnp
from jax import lax
from jax.experimental import pallas as pl
from jax.experimental.pallas import tpu as pltpu
from jax.experimental.pallas import tpu_sc as plsc

VOCAB = 1000000
NUM_PROMPT = 128
D = 32
TOTAL = 4096 * 200

NC, NS, L = 2, 16, 16
NW = NC * NS
PER_W = TOTAL // NW
BLK = 1024
NBLK = PER_W // BLK
GPER = 128
NG = BLK // GPER
DEPTH = 3


def _body(orig_hbm, new_hbm, idx_hbm, out_hbm, new_tab_v, *scr):
    idx_v = scr[0:DEPTH]
    safe_v = scr[DEPTH:2 * DEPTH]
    rows_v = scr[2 * DEPTH:3 * DEPTH]
    sg = scr[3 * DEPTH:4 * DEPTH]
    so = scr[4 * DEPTH:5 * DEPTH]
    si = scr[5 * DEPTH:6 * DEPTH]

    wid = lax.axis_index("s") * NC + lax.axis_index("c")
    wbase = wid * PER_W

    pltpu.sync_copy(new_hbm, new_tab_v)

    def clamp_block(s):
        def clamp(t, m):
            v = idx_v[s][pl.ds(t * L, L)]
            safe_v[s][pl.ds(t * L, L)] = jnp.minimum(v, VOCAB - 1)
            return jnp.maximum(m, v)
        m = lax.fori_loop(0, BLK // L, clamp, jnp.zeros((L,), jnp.int32))
        return plsc.all_reduce_population_count(m >= VOCAB)[0]

    def fix_block(s, hits):
        @pl.when(hits > 0)
        def _():
            def fix(t, c):
                v = idx_v[s][pl.ds(t * L, L)]
                cnt = plsc.all_reduce_population_count(v >= VOCAB)

                @pl.when(cnt[0] > 0)
                def _():
                    mask = v >= VOCAB
                    pidx = jnp.maximum(v - VOCAB, 0)
                    rowid = t * L + lax.iota(jnp.int32, L)
                    for dd in range(D):
                        dvec = jnp.full((L,), dd, jnp.int32)
                        vals = plsc.load_gather(new_tab_v, [pidx, dvec])
                        plsc.store_scatter(rows_v[s], [rowid, dvec], vals,
                                           mask=mask)
                return c
            lax.fori_loop(0, BLK // L, fix, 0)

    def fire_gathers(s):
        return [
            pltpu.async_copy(
                orig_hbm.at[safe_v[s].at[pl.ds(j * GPER, GPER)]],
                rows_v[s].at[pl.ds(j * GPER, GPER)],
                sg[s],
            )
            for j in range(NG)
        ]

    def finish_block(b):
        s = b % DEPTH
        for dsc in gather_descs[s]:
            dsc.wait()
        fix_block(s, hits_of[s])
        out_desc[s] = pltpu.async_copy(
            rows_v[s],
            out_hbm.at[pl.ds(wbase + b * BLK, BLK)],
            so[s],
        )

    out_desc = [None] * DEPTH
    gather_descs = [None] * DEPTH
    idx_desc = [None] * DEPTH
    hits_of = [None] * DEPTH

    idx_desc[0] = pltpu.async_copy(
        idx_hbm.at[pl.ds(wbase, BLK)], idx_v[0], si[0])

    for b in range(NBLK):
        s = b % DEPTH
        idx_desc[s].wait()
        hits_of[s] = clamp_block(s)
        if out_desc[s] is not None:
            out_desc[s].wait()
        gather_descs[s] = fire_gathers(s)
        if b >= DEPTH - 1:
            finish_block(b - (DEPTH - 1))
        if b + 1 < NBLK:
            sn = (b + 1) % DEPTH
            idx_desc[sn] = pltpu.async_copy(
                idx_hbm.at[pl.ds(wbase + (b + 1) * BLK, BLK)],
                idx_v[sn], si[sn])

    for b in range(NBLK - (DEPTH - 1), NBLK):
        finish_block(b)
    for dsc in out_desc:
        dsc.wait()


@functools.partial(jax.jit, static_argnames=())
def _lookup(orig_weight, new_weight, idx_flat):
    mesh = plsc.VectorSubcoreMesh(core_axis_name="c", subcore_axis_name="s")
    f = pl.kernel(
        _body,
        out_type=jax.ShapeDtypeStruct((TOTAL, D), jnp.float32),
        mesh=mesh,
        scratch_types=(
            [pltpu.VMEM((NUM_PROMPT, D), jnp.float32)]
            + [pltpu.VMEM((BLK,), jnp.int32) for _ in range(2 * DEPTH)]
            + [pltpu.VMEM((BLK, D), jnp.float32) for _ in range(DEPTH)]
            + [pltpu.SemaphoreType.DMA for _ in range(3 * DEPTH)]
        ),
        compiler_params=pltpu.CompilerParams(
            needs_layout_passes=False, use_tc_tiling_on_sc=False),
    )
    return f(orig_weight, new_weight, idx_flat)


def kernel(orig_weight, new_weight, input):
    idx_flat = input.reshape(-1).astype(jnp.int32)
    out = _lookup(orig_weight, new_weight, idx_flat)
    return out.reshape(input.shape + (D,))
